# baseline (speedup 1.0000x reference)
import functools

import jax
import jax.numpy as jnp
from jax import lax
from jax.experimental import pallas as pl
from jax.experimental.pallas import tpu as pltpu
from jax.experimental.pallas import tpu_sc as plsc

B, T, D = 1, 2048, 1024
H = 16
HD = D // H
HALF = HD // 2
E = 8
K = 2
INTER = 1024
SCALE = D ** (-0.5)

TQ = 256
TK = 256
TILE = 128
NTILES = (T * K + E * (TILE - 1)) // TILE + 1
NPOS = NTILES * TILE

NC, NS = 2, 16
NW = NC * NS

BF = jnp.bfloat16
F32 = jnp.float32


def _prelude_body(x_ref, g1_ref, w5_ref, b5_ref, q_ref, k_ref, v_ref):
    i = pl.program_id(0)
    xb = x_ref[...]
    ms = jnp.mean(xb * xb, axis=-1, keepdims=True)
    xn = (xb * lax.rsqrt(ms + 1e-6) * g1_ref[...]).astype(BF)
    proj = lax.dot_general(xn, w5_ref[...], (((1,), (1,)), ((), ())),
                           preferred_element_type=F32) + b5_ref[...]
    q, k, v = proj[:, :D], proj[:, D:2 * D], proj[:, 2 * D:3 * D]
    qs, ks = proj[:, 3 * D:4 * D], proj[:, 4 * D:]
    pos = (i * TQ
           + lax.broadcasted_iota(jnp.int32, (TQ, D), 0)).astype(F32)
    lane = lax.broadcasted_iota(jnp.int32, (TQ, D), 1) % HALF
    theta = jnp.exp(lane.astype(F32) * (-jnp.log(10000.0) / HALF))
    freq = pos * theta
    cosw = jnp.cos(freq)
    sinw = jnp.sin(freq)
    rq = ((q * cosw + qs * sinw) * SCALE).astype(BF)
    rk = (k * cosw + ks * sinw).astype(BF)
    vb = v.astype(BF)
    for h in range(H):
        sl = slice(h * HD, (h + 1) * HD)
        q_ref[h] = rq[:, sl]
        k_ref[h] = rk[:, sl]
        v_ref[h] = vb[:, sl]


def _attn_body(q_ref, k_ref, v_ref, o_ref):
    qi = pl.program_id(1)
    q = q_ref[0]

    def tile(j, carry):
        acc, l = carry
        ks = k_ref[0, pl.ds(j * TK, TK), :]
        vs = v_ref[0, pl.ds(j * TK, TK), :]
        s = lax.dot_general(q, ks, (((1,), (1,)), ((), ())),
                            preferred_element_type=F32)
        rows = qi * TQ + lax.broadcasted_iota(jnp.int32, (TQ, TK), 0)
        cols = j * TK + lax.broadcasted_iota(jnp.int32, (TQ, TK), 1)
        p = jnp.where(cols > rows, 0.0, jnp.exp(s))
        l = l + jnp.sum(p, axis=-1, keepdims=True)
        acc = acc + lax.dot_general(p.astype(BF), vs,
                                    (((1,), (0,)), ((), ())),
                                    preferred_element_type=F32)
        return acc, l

    acc0 = jnp.zeros((TQ, HD), F32)
    l0 = jnp.zeros((TQ, 1), F32)
    acc, l = lax.fori_loop(0, qi + 1, tile, (acc0, l0))
    o_ref[0] = (acc / l).astype(BF)


def _post_body(a_ref, x_ref, wo_ref, bo_ref, g2_ref, gw_ref,
               x1_ref, xn2_ref, xb_ref, rt_ref):
    a = a_ref[...]
    o = lax.dot_general(a, wo_ref[...], (((1,), (1,)), ((), ())),
                        preferred_element_type=F32)
    x1 = o + bo_ref[...] + x_ref[...]
    x1_ref[...] = x1
    ms = jnp.mean(x1 * x1, axis=-1, keepdims=True)
    xn2 = x1 * lax.rsqrt(ms + 1e-6) * g2_ref[...]
    xn2_ref[...] = xn2
    xb_ref[...] = xn2.astype(BF)
    lg = lax.dot_general(xn2, gw_ref[...], (((1,), (1,)), ((), ())),
                         preferred_element_type=F32)
    mx = jnp.max(lg, axis=-1, keepdims=True)
    ex = jnp.exp(lg - mx)
    p = ex / jnp.sum(ex, axis=-1, keepdims=True)
    colsE = lax.broadcasted_iota(jnp.int32, (TQ, E), 1)
    v1 = jnp.max(p, axis=-1, keepdims=True)
    i1 = jnp.min(jnp.where(p == v1, colsE, E), axis=-1, keepdims=True)
    p2 = jnp.where(colsE == i1, -1.0, p)
    v2 = jnp.max(p2, axis=-1, keepdims=True)
    i2 = jnp.min(jnp.where(p2 == v2, colsE, E), axis=-1, keepdims=True)
    rt_ref[...] = jnp.concatenate(
        [v1, v2, i1.astype(F32), i2.astype(F32),
         jnp.zeros((TQ, 4), F32)], axis=-1)


def _ffn_body(eids_ref, used_ref, xs_ref, w1_ref, w3_ref, w2_ref,
              b1_ref, b3_ref, b2_ref, wp_ref, o_ref):
    del eids_ref
    i = pl.program_id(0)

    @pl.when(used_ref[i] > 0)
    def _():
        xb = xs_ref[...]
        h1 = lax.dot_general(xb, w1_ref[0], (((1,), (1,)), ((), ())),
                             preferred_element_type=F32) + b1_ref[0]
        h3 = lax.dot_general(xb, w3_ref[0], (((1,), (1,)), ((), ())),
                             preferred_element_type=F32) + b3_ref[0]
        h = ((h1 * lax.logistic(h1)) * h3).astype(BF)
        o = lax.dot_general(h, w2_ref[0], (((1,), (1,)), ((), ())),
                            preferred_element_type=F32) + b2_ref[0]
        o_ref[...] = (o * wp_ref[:, 0:1]).astype(BF)


def _final_body(x1_ref, a_ref, b_ref, o_ref):
    o_ref[...] = (x1_ref[...] + a_ref[...].astype(F32)
                  + b_ref[...].astype(F32))


def _sc_mesh():
    return plsc.VectorSubcoreMesh(core_axis_name="c", subcore_axis_name="s")


def _sc_gather_rows(table_bf, indices, nrows):
    per_w = nrows // NW
    D2 = D // 2
    tab_i = lax.bitcast_convert_type(
        table_bf.reshape(table_bf.shape[0], D2, 2), jnp.int32)

    @functools.partial(
        pl.kernel,
        out_type=jax.ShapeDtypeStruct((nrows, D2), jnp.int32),
        mesh=_sc_mesh(),
        scratch_types=[
            pltpu.VMEM((per_w,), jnp.int32),
            pltpu.VMEM((per_w, D2), jnp.int32),
            pltpu.SemaphoreType.DMA,
        ],
    )
    def k(tab_hbm, idx_hbm, out_hbm, idx_v, rows_v, sem):
        wid = lax.axis_index("s") * NC + lax.axis_index("c")
        base = wid * per_w
        pltpu.sync_copy(idx_hbm.at[pl.ds(base, per_w)], idx_v)
        pltpu.async_copy(tab_hbm.at[idx_v], rows_v, sem).wait()
        pltpu.sync_copy(rows_v, out_hbm.at[pl.ds(base, per_w)])

    out_i = k(tab_i, indices)
    return lax.bitcast_convert_type(out_i, BF).reshape(nrows, D)


def _routing_meta(route):
    vals = route[:, :K]
    idx = route[:, K:2 * K].astype(jnp.int32)
    e_flat = idx.reshape(-1)
    val_flat = vals.reshape(-1)
    oh = (e_flat[:, None] == jnp.arange(E, dtype=jnp.int32)[None, :])
    oh = oh.astype(jnp.int32)
    ranks = jnp.cumsum(oh, axis=0) - oh
    r = jnp.sum(ranks * oh, axis=1)
    counts = jnp.sum(oh, axis=0)
    padded = ((counts + TILE - 1) // TILE) * TILE
    offs = jnp.concatenate(
        [jnp.zeros((1,), padded.dtype), jnp.cumsum(padded)[:-1]])
    P = (offs[e_flat] + r).astype(jnp.int32)
    tok_flat = jnp.arange(T * K, dtype=jnp.int32) // K
    tok_for_pos = jnp.zeros((NPOS,), jnp.int32).at[P].set(tok_flat)
    w_pos = jnp.zeros((NPOS,), F32).at[P].set(val_flat)
    cum = jnp.cumsum(padded)
    tile_starts = jnp.arange(NTILES, dtype=cum.dtype) * TILE
    eids = jnp.minimum(
        jnp.searchsorted(cum, tile_starts, side='right'), E - 1)
    eids = eids.astype(jnp.int32)
    used = (tile_starts < offs[eids] + counts[eids]).astype(jnp.int32)
    return tok_for_pos, w_pos, eids, used, P


def kernel(x, g1, g2, Wqkv, bqkv, Wout, bout, gateW, w1, b1, w2, b2, w3, b3):
    xf = x.reshape(T, D)
    Wg = Wqkv.reshape(H, 3, HD, D).transpose(1, 0, 2, 3)
    bg = bqkv.reshape(H, 3, HD).transpose(1, 0, 2)
    sgn = jnp.concatenate(
        [-jnp.ones((HALF, 1), F32), jnp.ones((HALF, 1), F32)])
    swp = jnp.concatenate([jnp.arange(HALF, HD), jnp.arange(0, HALF)])
    Wswap = Wg[:2, :, swp, :] * sgn
    bswap = bg[:2, :, swp] * sgn[:, 0]
    W5 = jnp.concatenate([Wg.reshape(3 * D, D),
                          Wswap.reshape(2 * D, D)]).astype(BF)
    b5 = jnp.concatenate([bg.reshape(3 * D),
                          bswap.reshape(2 * D)]).reshape(1, 5 * D)

    q4, k4, v4 = pl.pallas_call(
        _prelude_body,
        grid=(T // TQ,),
        in_specs=[
            pl.BlockSpec((TQ, D), lambda i: (i, 0)),
            pl.BlockSpec((1, D), lambda i: (0, 0)),
            pl.BlockSpec((5 * D, D), lambda i: (0, 0)),
            pl.BlockSpec((1, 5 * D), lambda i: (0, 0)),
        ],
        out_specs=[
            pl.BlockSpec((H, TQ, HD), lambda i: (0, i, 0)),
            pl.BlockSpec((H, TQ, HD), lambda i: (0, i, 0)),
            pl.BlockSpec((H, TQ, HD), lambda i: (0, i, 0)),
        ],
        out_shape=[
            jax.ShapeDtypeStruct((H, T, HD), BF),
            jax.ShapeDtypeStruct((H, T, HD), BF),
            jax.ShapeDtypeStruct((H, T, HD), BF),
        ],
    )(xf, g1.reshape(1, D), W5, b5)

    attn4 = pl.pallas_call(
        _attn_body,
        grid=(H, T // TQ),
        in_specs=[
            pl.BlockSpec((1, TQ, HD), lambda h, i: (h, i, 0)),
            pl.BlockSpec((1, T, HD), lambda h, i: (h, 0, 0)),
            pl.BlockSpec((1, T, HD), lambda h, i: (h, 0, 0)),
        ],
        out_specs=pl.BlockSpec((1, TQ, HD), lambda h, i: (h, i, 0)),
        out_shape=jax.ShapeDtypeStruct((H, T, HD), BF),
    )(q4, k4, v4)

    attn_t = attn4.transpose(1, 0, 2).reshape(T, D)

    x1, xn2, xn2b, route = pl.pallas_call(
        _post_body,
        grid=(T // TQ,),
        in_specs=[
            pl.BlockSpec((TQ, D), lambda i: (i, 0)),
            pl.BlockSpec((TQ, D), lambda i: (i, 0)),
            pl.BlockSpec((D, D), lambda i: (0, 0)),
            pl.BlockSpec((1, D), lambda i: (0, 0)),
            pl.BlockSpec((1, D), lambda i: (0, 0)),
            pl.BlockSpec((E, D), lambda i: (0, 0)),
        ],
        out_specs=[
            pl.BlockSpec((TQ, D), lambda i: (i, 0)),
            pl.BlockSpec((TQ, D), lambda i: (i, 0)),
            pl.BlockSpec((TQ, D), lambda i: (i, 0)),
            pl.BlockSpec((TQ, E), lambda i: (i, 0)),
        ],
        out_shape=[
            jax.ShapeDtypeStruct((T, D), F32),
            jax.ShapeDtypeStruct((T, D), F32),
            jax.ShapeDtypeStruct((T, D), BF),
            jax.ShapeDtypeStruct((T, E), F32),
        ],
    )(attn_t, xf, Wout.astype(BF), bout.reshape(1, D), g2.reshape(1, D),
      gateW)
    del xn2

    tok_for_pos, w_pos, eids, used, P = _routing_meta(route)

    xs = _sc_gather_rows(xn2b, tok_for_pos, NPOS)

    wp2 = jnp.broadcast_to(w_pos[:, None], (NPOS, 128))

    ys = pl.pallas_call(
        _ffn_body,
        grid_spec=pltpu.PrefetchScalarGridSpec(
            num_scalar_prefetch=2,
            grid=(NTILES,),
            in_specs=[
                pl.BlockSpec((TILE, D), lambda i, eids, used: (i, 0)),
                pl.BlockSpec((1, INTER, D),
                             lambda i, eids, used: (eids[i], 0, 0)),
                pl.BlockSpec((1, INTER, D),
                             lambda i, eids, used: (eids[i], 0, 0)),
                pl.BlockSpec((1, D, INTER),
                             lambda i, eids, used: (eids[i], 0, 0)),
                pl.BlockSpec((1, 1, INTER),
                             lambda i, eids, used: (eids[i], 0, 0)),
                pl.BlockSpec((1, 1, INTER),
                             lambda i, eids, used: (eids[i], 0, 0)),
                pl.BlockSpec((1, 1, D),
                             lambda i, eids, used: (eids[i], 0, 0)),
                pl.BlockSpec((TILE, 128), lambda i, eids, used: (i, 0)),
            ],
            out_specs=pl.BlockSpec((TILE, D), lambda i, eids, used: (i, 0)),
        ),
        out_shape=jax.ShapeDtypeStruct((NPOS, D), BF),
    )(eids, used, xs, w1.astype(BF), w3.astype(BF), w2.astype(BF),
      b1.reshape(E, 1, INTER), b3.reshape(E, 1, INTER), b2.reshape(E, 1, D),
      wp2)

    ab = _sc_gather_rows(ys, P.reshape(T, K).T.reshape(2 * T), 2 * T)

    out = pl.pallas_call(
        _final_body,
        grid=(T // TQ,),
        in_specs=[
            pl.BlockSpec((TQ, D), lambda i: (i, 0)),
            pl.BlockSpec((TQ, D), lambda i: (i, 0)),
            pl.BlockSpec((TQ, D), lambda i: (i + T // TQ, 0)),
        ],
        out_specs=pl.BlockSpec((TQ, D), lambda i: (i, 0)),
        out_shape=jax.ShapeDtypeStruct((T, D), F32),
    )(x1, ab, ab)
    return out.reshape(B, T, D)


# speedup vs baseline: 1.6078x; 1.6078x over previous
import functools

import jax
import jax.numpy as jnp
from jax import lax
from jax.experimental import pallas as pl
from jax.experimental.pallas import tpu as pltpu
from jax.experimental.pallas import tpu_sc as plsc

B, T, D = 1, 2048, 1024
H = 16
HD = D // H
HALF = HD // 2
E = 8
K = 2
INTER = 1024
SCALE = D ** (-0.5)

TQ = 256
TK = 256
NKT = T // TK
TILE = 128
NTILES = (T * K + E * (TILE - 1)) // TILE + 1
NPOS = NTILES * TILE

NC, NS = 2, 16
NW = NC * NS

BF = jnp.bfloat16
F32 = jnp.float32


def _prelude_body(x_ref, g1_ref, w5_ref, b5_ref, q_ref, k_ref, v_ref):
    i = pl.program_id(0)
    xb = x_ref[...]
    ms = jnp.mean(xb * xb, axis=-1, keepdims=True)
    xn = (xb * lax.rsqrt(ms + 1e-6) * g1_ref[...]).astype(BF)
    proj = lax.dot_general(xn, w5_ref[...], (((1,), (1,)), ((), ())),
                           preferred_element_type=F32) + b5_ref[...]
    q, k, v = proj[:, :D], proj[:, D:2 * D], proj[:, 2 * D:3 * D]
    qs, ks = proj[:, 3 * D:4 * D], proj[:, 4 * D:]
    pos = (i * TQ
           + lax.broadcasted_iota(jnp.int32, (TQ, 128), 0)).astype(F32)
    lane = lax.broadcasted_iota(jnp.int32, (TQ, 128), 1) % HALF
    theta = jnp.exp(lane.astype(F32) * (-jnp.log(10000.0) / HALF))
    freq = pos * theta
    cosw = jnp.tile(jnp.cos(freq), (1, D // 128))
    sinw = jnp.tile(jnp.sin(freq), (1, D // 128))
    rq = ((q * cosw + qs * sinw) * SCALE).astype(BF)
    rk = (k * cosw + ks * sinw).astype(BF)
    vb = v.astype(BF)
    for h in range(H):
        sl = slice(h * HD, (h + 1) * HD)
        q_ref[h] = rq[:, sl]
        k_ref[h] = rk[:, sl]
        v_ref[h] = vb[:, sl]


def _attn_body(q_ref, k_ref, v_ref, o_ref, acc_ref, l_ref):
    qi = pl.program_id(1)
    acc_ref[...] = jnp.zeros_like(acc_ref)
    l_ref[...] = jnp.zeros_like(l_ref)
    rows = qi * TQ + lax.broadcasted_iota(jnp.int32, (TQ, TK), 0)
    for j in range(NKT):
        @pl.when(j <= qi)
        def _():
            cols = j * TK + lax.broadcasted_iota(jnp.int32, (TQ, TK), 1)
            future = cols > rows
            for s_idx in range(2):
                q = q_ref[s_idx]
                ks = k_ref[s_idx, pl.ds(j * TK, TK), :]
                vs = v_ref[s_idx, pl.ds(j * TK, TK), :]
                s = lax.dot_general(q, ks, (((1,), (1,)), ((), ())),
                                    preferred_element_type=F32)
                p = jnp.where(future, 0.0, jnp.exp(s))
                l_ref[:, s_idx:s_idx + 1] = (
                    l_ref[:, s_idx:s_idx + 1]
                    + jnp.sum(p, axis=-1, keepdims=True))
                av = lax.dot_general(p.astype(BF), vs,
                                     (((1,), (0,)), ((), ())),
                                     preferred_element_type=F32)
                sl = slice(s_idx * HD, (s_idx + 1) * HD)
                acc_ref[:, sl] = acc_ref[:, sl] + av
    for s_idx in range(2):
        sl = slice(s_idx * HD, (s_idx + 1) * HD)
        o_ref[:, sl] = ((acc_ref[:, sl] / l_ref[:, s_idx:s_idx + 1])
                        .astype(BF))


def _post_body(a_ref, x_ref, wo_ref, bo_ref, g2_ref, gw_ref,
               x1_ref, xn2_ref, rt_ref):
    a = a_ref[...]
    o = lax.dot_general(a, wo_ref[...], (((1,), (1,)), ((), ())),
                        preferred_element_type=F32)
    x1 = o + bo_ref[...] + x_ref[...]
    x1_ref[...] = x1
    ms = jnp.mean(x1 * x1, axis=-1, keepdims=True)
    xn2 = x1 * lax.rsqrt(ms + 1e-6) * g2_ref[...]
    xn2_ref[...] = xn2
    lg = lax.dot_general(xn2, gw_ref[...], (((1,), (1,)), ((), ())),
                         preferred_element_type=F32)
    mx = jnp.max(lg, axis=-1, keepdims=True)
    ex = jnp.exp(lg - mx)
    p = ex / jnp.sum(ex, axis=-1, keepdims=True)
    colsE = lax.broadcasted_iota(jnp.int32, (TQ, E), 1)
    v1 = jnp.max(p, axis=-1, keepdims=True)
    i1 = jnp.min(jnp.where(p == v1, colsE, E), axis=-1, keepdims=True)
    p2 = jnp.where(colsE == i1, -1.0, p)
    v2 = jnp.max(p2, axis=-1, keepdims=True)
    i2 = jnp.min(jnp.where(p2 == v2, colsE, E), axis=-1, keepdims=True)
    rt_ref[...] = jnp.concatenate(
        [v1, v2, i1.astype(F32), i2.astype(F32),
         jnp.zeros((TQ, 4), F32)], axis=-1)


def _ffn_body(eids_ref, used_ref, xs_ref, w1_ref, w3_ref, w2_ref,
              b1_ref, b3_ref, b2_ref, wp_ref, o_ref):
    del eids_ref
    i = pl.program_id(0)

    @pl.when(used_ref[i] > 0)
    def _():
        xb = xs_ref[...].astype(BF)
        h1 = lax.dot_general(xb, w1_ref[0], (((1,), (1,)), ((), ())),
                             preferred_element_type=F32) + b1_ref[0]
        h3 = lax.dot_general(xb, w3_ref[0], (((1,), (1,)), ((), ())),
                             preferred_element_type=F32) + b3_ref[0]
        h = ((h1 * lax.logistic(h1)) * h3).astype(BF)
        o = lax.dot_general(h, w2_ref[0], (((1,), (1,)), ((), ())),
                            preferred_element_type=F32) + b2_ref[0]
        o_ref[...] = o * wp_ref[:, 0:1]


def _final_body(x1_ref, a_ref, b_ref, o_ref):
    o_ref[...] = x1_ref[...] + a_ref[...] + b_ref[...]


def _sc_mesh():
    return plsc.VectorSubcoreMesh(core_axis_name="c", subcore_axis_name="s")


def _sc_gather_rows(table, indices, nrows, chunk):
    per_w = nrows // NW

    @functools.partial(
        pl.kernel,
        out_type=jax.ShapeDtypeStruct((nrows, D), F32),
        mesh=_sc_mesh(),
        scratch_types=[
            pltpu.VMEM((chunk,), jnp.int32),
            pltpu.VMEM((chunk, D), F32),
            pltpu.SemaphoreType.DMA,
        ],
    )
    def k(tab_hbm, idx_hbm, out_hbm, idx_v, rows_v, sem):
        wid = lax.axis_index("s") * NC + lax.axis_index("c")
        base = wid * per_w

        @pl.loop(0, per_w // chunk)
        def _(c):
            off = base + c * chunk
            pltpu.sync_copy(idx_hbm.at[pl.ds(off, chunk)], idx_v)
            pltpu.async_copy(tab_hbm.at[idx_v], rows_v, sem).wait()
            pltpu.sync_copy(rows_v, out_hbm.at[pl.ds(off, chunk)])

    return k(table, indices)


def _routing_meta(route):
    vals = route[:, :K]
    idx = route[:, K:2 * K].astype(jnp.int32)
    e_flat = idx.reshape(-1)
    val_flat = vals.reshape(-1)
    oh = (e_flat[:, None] == jnp.arange(E, dtype=jnp.int32)[None, :])
    oh = oh.astype(jnp.int32)
    ranks = jnp.cumsum(oh, axis=0) - oh
    r = jnp.sum(ranks * oh, axis=1)
    counts = jnp.sum(oh, axis=0)
    padded = ((counts + TILE - 1) // TILE) * TILE
    offs = jnp.concatenate(
        [jnp.zeros((1,), padded.dtype), jnp.cumsum(padded)[:-1]])
    P = (offs[e_flat] + r).astype(jnp.int32)
    tok_flat = jnp.arange(T * K, dtype=jnp.int32) // K
    tok_for_pos = jnp.zeros((NPOS,), jnp.int32).at[P].set(tok_flat)
    w_pos = jnp.zeros((NPOS,), F32).at[P].set(val_flat)
    cum = jnp.cumsum(padded)
    tile_starts = jnp.arange(NTILES, dtype=cum.dtype) * TILE
    eids = jnp.minimum(
        jnp.searchsorted(cum, tile_starts, side='right'), E - 1)
    eids = eids.astype(jnp.int32)
    used = (tile_starts < offs[eids] + counts[eids]).astype(jnp.int32)
    return tok_for_pos, w_pos, eids, used, P


def kernel(x, g1, g2, Wqkv, bqkv, Wout, bout, gateW, w1, b1, w2, b2, w3, b3):
    xf = x.reshape(T, D)
    Wg = Wqkv.reshape(H, 3, HD, D).transpose(1, 0, 2, 3)
    bg = bqkv.reshape(H, 3, HD).transpose(1, 0, 2)
    sgn = jnp.concatenate(
        [-jnp.ones((HALF, 1), F32), jnp.ones((HALF, 1), F32)])
    swp = jnp.concatenate([jnp.arange(HALF, HD), jnp.arange(0, HALF)])
    Wswap = Wg[:2, :, swp, :] * sgn
    bswap = bg[:2, :, swp] * sgn[:, 0]
    W5 = jnp.concatenate([Wg.reshape(3 * D, D),
                          Wswap.reshape(2 * D, D)]).astype(BF)
    b5 = jnp.concatenate([bg.reshape(3 * D),
                          bswap.reshape(2 * D)]).reshape(1, 5 * D)

    q4, k4, v4 = pl.pallas_call(
        _prelude_body,
        grid=(T // TQ,),
        in_specs=[
            pl.BlockSpec((TQ, D), lambda i: (i, 0)),
            pl.BlockSpec((1, D), lambda i: (0, 0)),
            pl.BlockSpec((5 * D, D), lambda i: (0, 0)),
            pl.BlockSpec((1, 5 * D), lambda i: (0, 0)),
        ],
        out_specs=[
            pl.BlockSpec((H, TQ, HD), lambda i: (0, i, 0)),
            pl.BlockSpec((H, TQ, HD), lambda i: (0, i, 0)),
            pl.BlockSpec((H, TQ, HD), lambda i: (0, i, 0)),
        ],
        out_shape=[
            jax.ShapeDtypeStruct((H, T, HD), BF),
            jax.ShapeDtypeStruct((H, T, HD), BF),
            jax.ShapeDtypeStruct((H, T, HD), BF),
        ],
    )(xf, g1.reshape(1, D), W5, b5)

    attn_t = pl.pallas_call(
        _attn_body,
        grid=(H // 2, T // TQ),
        in_specs=[
            pl.BlockSpec((2, TQ, HD), lambda h, i: (h, i, 0)),
            pl.BlockSpec((2, T, HD), lambda h, i: (h, 0, 0)),
            pl.BlockSpec((2, T, HD), lambda h, i: (h, 0, 0)),
        ],
        out_specs=pl.BlockSpec((TQ, 2 * HD), lambda h, i: (i, h)),
        out_shape=jax.ShapeDtypeStruct((T, D), BF),
        scratch_shapes=[
            pltpu.VMEM((TQ, 2 * HD), F32),
            pltpu.VMEM((TQ, 128), F32),
        ],
    )(q4, k4, v4)

    x1, xn2, route = pl.pallas_call(
        _post_body,
        grid=(T // TQ,),
        in_specs=[
            pl.BlockSpec((TQ, D), lambda i: (i, 0)),
            pl.BlockSpec((TQ, D), lambda i: (i, 0)),
            pl.BlockSpec((D, D), lambda i: (0, 0)),
            pl.BlockSpec((1, D), lambda i: (0, 0)),
            pl.BlockSpec((1, D), lambda i: (0, 0)),
            pl.BlockSpec((E, D), lambda i: (0, 0)),
        ],
        out_specs=[
            pl.BlockSpec((TQ, D), lambda i: (i, 0)),
            pl.BlockSpec((TQ, D), lambda i: (i, 0)),
            pl.BlockSpec((TQ, E), lambda i: (i, 0)),
        ],
        out_shape=[
            jax.ShapeDtypeStruct((T, D), F32),
            jax.ShapeDtypeStruct((T, D), F32),
            jax.ShapeDtypeStruct((T, E), F32),
        ],
    )(attn_t, xf, Wout.astype(BF), bout.reshape(1, D), g2.reshape(1, D),
      gateW)

    tok_for_pos, w_pos, eids, used, P = _routing_meta(route)

    xs = _sc_gather_rows(xn2, tok_for_pos, NPOS, 80)

    wp8 = jnp.broadcast_to(w_pos[:, None], (NPOS, 8))

    ys = pl.pallas_call(
        _ffn_body,
        grid_spec=pltpu.PrefetchScalarGridSpec(
            num_scalar_prefetch=2,
            grid=(NTILES,),
            in_specs=[
                pl.BlockSpec((TILE, D), lambda i, eids, used: (i, 0)),
                pl.BlockSpec((1, INTER, D),
                             lambda i, eids, used: (eids[i], 0, 0)),
                pl.BlockSpec((1, INTER, D),
                             lambda i, eids, used: (eids[i], 0, 0)),
                pl.BlockSpec((1, D, INTER),
                             lambda i, eids, used: (eids[i], 0, 0)),
                pl.BlockSpec((1, 1, INTER),
                             lambda i, eids, used: (eids[i], 0, 0)),
                pl.BlockSpec((1, 1, INTER),
                             lambda i, eids, used: (eids[i], 0, 0)),
                pl.BlockSpec((1, 1, D),
                             lambda i, eids, used: (eids[i], 0, 0)),
                pl.BlockSpec((TILE, 8), lambda i, eids, used: (i, 0)),
            ],
            out_specs=pl.BlockSpec((TILE, D), lambda i, eids, used: (i, 0)),
        ),
        out_shape=jax.ShapeDtypeStruct((NPOS, D), F32),
    )(eids, used, xs, w1.astype(BF), w3.astype(BF), w2.astype(BF),
      b1.reshape(E, 1, INTER), b3.reshape(E, 1, INTER), b2.reshape(E, 1, D),
      wp8)

    ab = _sc_gather_rows(ys, P.reshape(T, K).T.reshape(2 * T), 2 * T, 64)

    out = pl.pallas_call(
        _final_body,
        grid=(T // TQ,),
        in_specs=[
            pl.BlockSpec((TQ, D), lambda i: (i, 0)),
            pl.BlockSpec((TQ, D), lambda i: (i, 0)),
            pl.BlockSpec((TQ, D), lambda i: (i + T // TQ, 0)),
        ],
        out_specs=pl.BlockSpec((TQ, D), lambda i: (i, 0)),
        out_shape=jax.ShapeDtypeStruct((T, D), F32),
    )(x1, ab, ab)
    return out.reshape(B, T, D)


# speedup vs baseline: 1.8687x; 1.1623x over previous
import functools

import jax
import jax.numpy as jnp
from jax import lax
from jax.experimental import pallas as pl
from jax.experimental.pallas import tpu as pltpu
from jax.experimental.pallas import tpu_sc as plsc

B, T, D = 1, 2048, 1024
H = 16
HD = D // H
HALF = HD // 2
E = 8
K = 2
INTER = 1024
SCALE = D ** (-0.5)

TQ = 256
TK = 256
NKT = T // TK
TILE = 128
NTILES = (T * K + E * (TILE - 1)) // TILE + 1
NPOS = NTILES * TILE

NC, NS = 2, 16
NW = NC * NS

BF = jnp.bfloat16
F32 = jnp.float32


def _prelude_body(x_ref, g1_ref, w5_ref, b5_ref, q_ref, k_ref, v_ref,
                  w5b_ref):
    i = pl.program_id(0)

    @pl.when(i == 0)
    def _():
        w5b_ref[...] = w5_ref[...].astype(BF)

    xb = x_ref[...]
    ms = jnp.mean(xb * xb, axis=-1, keepdims=True)
    xn = (xb * lax.rsqrt(ms + 1e-6) * g1_ref[...]).astype(BF)
    proj = lax.dot_general(xn, w5b_ref[...], (((1,), (1,)), ((), ())),
                           preferred_element_type=F32) + b5_ref[...]
    q, k, v = proj[:, :D], proj[:, D:2 * D], proj[:, 2 * D:3 * D]
    qs, ks = proj[:, 3 * D:4 * D], proj[:, 4 * D:]
    pos = (i * TQ
           + lax.broadcasted_iota(jnp.int32, (TQ, 128), 0)).astype(F32)
    lane = lax.broadcasted_iota(jnp.int32, (TQ, 128), 1) % HALF
    theta = jnp.exp(lane.astype(F32) * (-jnp.log(10000.0) / HALF))
    freq = pos * theta
    cosw = jnp.tile(jnp.cos(freq), (1, D // 128))
    sinw = jnp.tile(jnp.sin(freq), (1, D // 128))
    rq = ((q * cosw + qs * sinw) * SCALE).astype(BF)
    rk = (k * cosw + ks * sinw).astype(BF)
    vb = v.astype(BF)
    for h in range(H):
        sl = slice(h * HD, (h + 1) * HD)
        q_ref[h] = rq[:, sl]
        k_ref[h] = rk[:, sl]
        v_ref[h] = vb[:, sl]


def _attn_body(q_ref, k_ref, v_ref, o_ref, acc_ref, l_ref):
    qi = pl.program_id(1)
    acc_ref[...] = jnp.zeros_like(acc_ref)
    l_ref[...] = jnp.zeros_like(l_ref)
    rows = qi * TQ + lax.broadcasted_iota(jnp.int32, (TQ, TK), 0)
    for j in range(NKT):
        @pl.when(j <= qi)
        def _():
            cols = j * TK + lax.broadcasted_iota(jnp.int32, (TQ, TK), 1)
            future = cols > rows
            for s_idx in range(2):
                q = q_ref[s_idx]
                ks = k_ref[s_idx, pl.ds(j * TK, TK), :]
                vs = v_ref[s_idx, pl.ds(j * TK, TK), :]
                s = lax.dot_general(q, ks, (((1,), (1,)), ((), ())),
                                    preferred_element_type=F32)
                p = jnp.where(future, 0.0, jnp.exp(s))
                l_ref[:, s_idx:s_idx + 1] = (
                    l_ref[:, s_idx:s_idx + 1]
                    + jnp.sum(p, axis=-1, keepdims=True))
                av = lax.dot_general(p.astype(BF), vs,
                                     (((1,), (0,)), ((), ())),
                                     preferred_element_type=F32)
                sl = slice(s_idx * HD, (s_idx + 1) * HD)
                acc_ref[:, sl] = acc_ref[:, sl] + av
    for s_idx in range(2):
        sl = slice(s_idx * HD, (s_idx + 1) * HD)
        o_ref[:, sl] = ((acc_ref[:, sl] / l_ref[:, s_idx:s_idx + 1])
                        .astype(BF))


def _post_body(a_ref, x_ref, wo_ref, bo_ref, g2_ref, gw_ref,
               x1_ref, xn2_ref, rt_ref, wob_ref):
    i = pl.program_id(0)

    @pl.when(i == 0)
    def _():
        wob_ref[...] = wo_ref[...].astype(BF)

    a = a_ref[...]
    o = lax.dot_general(a, wob_ref[...], (((1,), (1,)), ((), ())),
                        preferred_element_type=F32)
    x1 = o + bo_ref[...] + x_ref[...]
    x1_ref[...] = x1
    ms = jnp.mean(x1 * x1, axis=-1, keepdims=True)
    xn2 = x1 * lax.rsqrt(ms + 1e-6) * g2_ref[...]
    xn2_ref[...] = xn2
    lg = lax.dot_general(xn2, gw_ref[...], (((1,), (1,)), ((), ())),
                         preferred_element_type=F32)
    mx = jnp.max(lg, axis=-1, keepdims=True)
    ex = jnp.exp(lg - mx)
    p = ex / jnp.sum(ex, axis=-1, keepdims=True)
    colsE = lax.broadcasted_iota(jnp.int32, (TQ, E), 1)
    v1 = jnp.max(p, axis=-1, keepdims=True)
    i1 = jnp.min(jnp.where(p == v1, colsE, E), axis=-1, keepdims=True)
    p2 = jnp.where(colsE == i1, -1.0, p)
    v2 = jnp.max(p2, axis=-1, keepdims=True)
    i2 = jnp.min(jnp.where(p2 == v2, colsE, E), axis=-1, keepdims=True)
    rt_ref[...] = jnp.concatenate(
        [v1, v2, i1.astype(F32), i2.astype(F32),
         jnp.zeros((TQ, 4), F32)], axis=-1)


def _ffn_body(eids_ref, used_ref, xs_ref, w1_ref, w3_ref, w2_ref,
              b1_ref, b3_ref, b2_ref, wp_ref, o_ref,
              w1b_ref, w3b_ref, w2b_ref):
    i = pl.program_id(0)
    prev = eids_ref[jnp.maximum(i - 1, 0)]
    changed = jnp.logical_or(i == 0, eids_ref[i] != prev)

    @pl.when(changed)
    def _():
        w1b_ref[...] = w1_ref[0].astype(BF)
        w3b_ref[...] = w3_ref[0].astype(BF)
        w2b_ref[...] = w2_ref[0].astype(BF)

    @pl.when(used_ref[i] > 0)
    def _():
        xb = xs_ref[...].astype(BF)
        h1 = lax.dot_general(xb, w1b_ref[...], (((1,), (1,)), ((), ())),
                             preferred_element_type=F32) + b1_ref[0]
        h3 = lax.dot_general(xb, w3b_ref[...], (((1,), (1,)), ((), ())),
                             preferred_element_type=F32) + b3_ref[0]
        h = ((h1 * lax.logistic(h1)) * h3).astype(BF)
        o = lax.dot_general(h, w2b_ref[...], (((1,), (1,)), ((), ())),
                            preferred_element_type=F32) + b2_ref[0]
        o_ref[...] = o * wp_ref[:, 0:1]


def _final_body(x1_ref, a_ref, b_ref, o_ref):
    o_ref[...] = x1_ref[...] + a_ref[...] + b_ref[...]


def _sc_mesh():
    return plsc.VectorSubcoreMesh(core_axis_name="c", subcore_axis_name="s")


def _sc_gather_rows(table, indices, nrows, chunk):
    per_w = nrows // NW

    @functools.partial(
        pl.kernel,
        out_type=jax.ShapeDtypeStruct((nrows, D), F32),
        mesh=_sc_mesh(),
        scratch_types=[
            pltpu.VMEM((chunk,), jnp.int32),
            pltpu.VMEM((chunk, D), F32),
            pltpu.SemaphoreType.DMA,
        ],
    )
    def k(tab_hbm, idx_hbm, out_hbm, idx_v, rows_v, sem):
        wid = lax.axis_index("s") * NC + lax.axis_index("c")
        base = wid * per_w

        @pl.loop(0, per_w // chunk)
        def _(c):
            off = base + c * chunk
            pltpu.sync_copy(idx_hbm.at[pl.ds(off, chunk)], idx_v)
            pltpu.async_copy(tab_hbm.at[idx_v], rows_v, sem).wait()
            pltpu.sync_copy(rows_v, out_hbm.at[pl.ds(off, chunk)])

    return k(table, indices)


def _routing_meta(route):
    vals = route[:, :K]
    idx = route[:, K:2 * K].astype(jnp.int32)
    e_flat = idx.reshape(-1)
    val_flat = vals.reshape(-1)
    oh = (e_flat[:, None] == jnp.arange(E, dtype=jnp.int32)[None, :])
    oh = oh.astype(jnp.int32)
    z = oh
    sh = 1
    while sh < T * K:
        z = z + jnp.pad(z, ((sh, 0), (0, 0)))[:T * K]
        sh *= 2
    ranks = z - oh
    r = jnp.sum(ranks * oh, axis=1)
    counts = jnp.sum(oh, axis=0)
    padded = ((counts + TILE - 1) // TILE) * TILE
    offs = jnp.concatenate(
        [jnp.zeros((1,), padded.dtype), jnp.cumsum(padded)[:-1]])
    P = (offs[e_flat] + r).astype(jnp.int32)
    tok_flat = jnp.arange(T * K, dtype=jnp.int32) // K
    tok_for_pos = (jnp.arange(NPOS, dtype=jnp.int32) % T).at[P].set(tok_flat)
    w_pos = jnp.zeros((NPOS,), F32).at[P].set(val_flat)
    cum = jnp.cumsum(padded)
    tile_starts = jnp.arange(NTILES, dtype=cum.dtype) * TILE
    eids = jnp.minimum(
        jnp.searchsorted(cum, tile_starts, side='right'), E - 1)
    eids = eids.astype(jnp.int32)
    used = (tile_starts < offs[eids] + counts[eids]).astype(jnp.int32)
    return tok_for_pos, w_pos, eids, used, P


def kernel(x, g1, g2, Wqkv, bqkv, Wout, bout, gateW, w1, b1, w2, b2, w3, b3):
    xf = x.reshape(T, D)
    Wg = Wqkv.reshape(H, 3, HD, D).transpose(1, 0, 2, 3)
    bg = bqkv.reshape(H, 3, HD).transpose(1, 0, 2)
    sgn = jnp.concatenate(
        [-jnp.ones((HALF, 1), F32), jnp.ones((HALF, 1), F32)])
    swp = jnp.concatenate([jnp.arange(HALF, HD), jnp.arange(0, HALF)])
    Wswap = Wg[:2, :, swp, :] * sgn
    bswap = bg[:2, :, swp] * sgn[:, 0]
    W5 = jnp.concatenate([Wg.reshape(3 * D, D),
                          Wswap.reshape(2 * D, D)])
    b5 = jnp.concatenate([bg.reshape(3 * D),
                          bswap.reshape(2 * D)]).reshape(1, 5 * D)

    q4, k4, v4 = pl.pallas_call(
        _prelude_body,
        grid=(T // TQ,),
        in_specs=[
            pl.BlockSpec((TQ, D), lambda i: (i, 0)),
            pl.BlockSpec((1, D), lambda i: (0, 0)),
            pl.BlockSpec((5 * D, D), lambda i: (0, 0)),
            pl.BlockSpec((1, 5 * D), lambda i: (0, 0)),
        ],
        out_specs=[
            pl.BlockSpec((H, TQ, HD), lambda i: (0, i, 0)),
            pl.BlockSpec((H, TQ, HD), lambda i: (0, i, 0)),
            pl.BlockSpec((H, TQ, HD), lambda i: (0, i, 0)),
        ],
        out_shape=[
            jax.ShapeDtypeStruct((H, T, HD), BF),
            jax.ShapeDtypeStruct((H, T, HD), BF),
            jax.ShapeDtypeStruct((H, T, HD), BF),
        ],
        scratch_shapes=[pltpu.VMEM((5 * D, D), BF)],
    )(xf, g1.reshape(1, D), W5, b5)

    attn_t = pl.pallas_call(
        _attn_body,
        grid=(H // 2, T // TQ),
        in_specs=[
            pl.BlockSpec((2, TQ, HD), lambda h, i: (h, i, 0)),
            pl.BlockSpec((2, T, HD), lambda h, i: (h, 0, 0)),
            pl.BlockSpec((2, T, HD), lambda h, i: (h, 0, 0)),
        ],
        out_specs=pl.BlockSpec((TQ, 2 * HD), lambda h, i: (i, h)),
        out_shape=jax.ShapeDtypeStruct((T, D), BF),
        scratch_shapes=[
            pltpu.VMEM((TQ, 2 * HD), F32),
            pltpu.VMEM((TQ, 128), F32),
        ],
    )(q4, k4, v4)

    x1, xn2, route = pl.pallas_call(
        _post_body,
        grid=(T // TQ,),
        in_specs=[
            pl.BlockSpec((TQ, D), lambda i: (i, 0)),
            pl.BlockSpec((TQ, D), lambda i: (i, 0)),
            pl.BlockSpec((D, D), lambda i: (0, 0)),
            pl.BlockSpec((1, D), lambda i: (0, 0)),
            pl.BlockSpec((1, D), lambda i: (0, 0)),
            pl.BlockSpec((E, D), lambda i: (0, 0)),
        ],
        out_specs=[
            pl.BlockSpec((TQ, D), lambda i: (i, 0)),
            pl.BlockSpec((TQ, D), lambda i: (i, 0)),
            pl.BlockSpec((TQ, E), lambda i: (i, 0)),
        ],
        out_shape=[
            jax.ShapeDtypeStruct((T, D), F32),
            jax.ShapeDtypeStruct((T, D), F32),
            jax.ShapeDtypeStruct((T, E), F32),
        ],
        scratch_shapes=[pltpu.VMEM((D, D), BF)],
    )(attn_t, xf, Wout, bout.reshape(1, D), g2.reshape(1, D),
      gateW)

    tok_for_pos, w_pos, eids, used, P = _routing_meta(route)

    xs = _sc_gather_rows(xn2, tok_for_pos, NPOS, 80)

    wp8 = jnp.broadcast_to(w_pos[:, None], (NPOS, 8))

    ys = pl.pallas_call(
        _ffn_body,
        grid_spec=pltpu.PrefetchScalarGridSpec(
            num_scalar_prefetch=2,
            grid=(NTILES,),
            in_specs=[
                pl.BlockSpec((TILE, D), lambda i, eids, used: (i, 0)),
                pl.BlockSpec((1, INTER, D),
                             lambda i, eids, used: (eids[i], 0, 0)),
                pl.BlockSpec((1, INTER, D),
                             lambda i, eids, used: (eids[i], 0, 0)),
                pl.BlockSpec((1, D, INTER),
                             lambda i, eids, used: (eids[i], 0, 0)),
                pl.BlockSpec((1, 1, INTER),
                             lambda i, eids, used: (eids[i], 0, 0)),
                pl.BlockSpec((1, 1, INTER),
                             lambda i, eids, used: (eids[i], 0, 0)),
                pl.BlockSpec((1, 1, D),
                             lambda i, eids, used: (eids[i], 0, 0)),
                pl.BlockSpec((TILE, 8), lambda i, eids, used: (i, 0)),
            ],
            out_specs=pl.BlockSpec((TILE, D), lambda i, eids, used: (i, 0)),
            scratch_shapes=[
                pltpu.VMEM((INTER, D), BF),
                pltpu.VMEM((INTER, D), BF),
                pltpu.VMEM((D, INTER), BF),
            ],
        ),
        out_shape=jax.ShapeDtypeStruct((NPOS, D), F32),
    )(eids, used, xs, w1, w3, w2,
      b1.reshape(E, 1, INTER), b3.reshape(E, 1, INTER), b2.reshape(E, 1, D),
      wp8)

    ab = _sc_gather_rows(ys, P.reshape(T, K).T.reshape(2 * T), 2 * T, 64)

    out = pl.pallas_call(
        _final_body,
        grid=(T // TQ,),
        in_specs=[
            pl.BlockSpec((TQ, D), lambda i: (i, 0)),
            pl.BlockSpec((TQ, D), lambda i: (i, 0)),
            pl.BlockSpec((TQ, D), lambda i: (i + T // TQ, 0)),
        ],
        out_specs=pl.BlockSpec((TQ, D), lambda i: (i, 0)),
        out_shape=jax.ShapeDtypeStruct((T, D), F32),
    )(x1, ab, ab)
    return out.reshape(B, T, D)


# speedup vs baseline: 2.3404x; 1.2524x over previous
import functools

import jax
import jax.numpy as jnp
from jax import lax
from jax.experimental import pallas as pl
from jax.experimental.pallas import tpu as pltpu
from jax.experimental.pallas import tpu_sc as plsc

B, T, D = 1, 2048, 1024
H = 16
HD = D // H
HALF = HD // 2
E = 8
K = 2
INTER = 1024
SCALE = D ** (-0.5)

TQ = 256
TK = 256
NKT = T // TK
TILE = 128
NTILES = (T * K + E * (TILE - 1)) // TILE + 1
NPOS = NTILES * TILE

NC, NS = 2, 16
NW = NC * NS

BF = jnp.bfloat16
F32 = jnp.float32


def _prelude_body(x_ref, g1_ref, w5_ref, b5_ref, q_ref, k_ref, v_ref,
                  w5b_ref):
    i = pl.program_id(0)

    @pl.when(i == 0)
    def _():
        w5b_ref[...] = w5_ref[...].astype(BF)

    xb = x_ref[...]
    ms = jnp.mean(xb * xb, axis=-1, keepdims=True)
    xn = (xb * lax.rsqrt(ms + 1e-6) * g1_ref[...]).astype(BF)
    proj = lax.dot_general(xn, w5b_ref[...], (((1,), (1,)), ((), ())),
                           preferred_element_type=F32) + b5_ref[...]
    q, k, v = proj[:, :D], proj[:, D:2 * D], proj[:, 2 * D:3 * D]
    qs, ks = proj[:, 3 * D:4 * D], proj[:, 4 * D:]
    pos = (i * TQ
           + lax.broadcasted_iota(jnp.int32, (TQ, 128), 0)).astype(F32)
    lane = lax.broadcasted_iota(jnp.int32, (TQ, 128), 1) % HALF
    theta = jnp.exp(lane.astype(F32) * (-jnp.log(10000.0) / HALF))
    freq = pos * theta
    cosw = jnp.tile(jnp.cos(freq), (1, D // 128))
    sinw = jnp.tile(jnp.sin(freq), (1, D // 128))
    rq = ((q * cosw + qs * sinw) * SCALE).astype(BF)
    rk = (k * cosw + ks * sinw).astype(BF)
    vb = v.astype(BF)
    for h in range(H):
        sl = slice(h * HD, (h + 1) * HD)
        q_ref[h] = rq[:, sl]
        k_ref[h] = rk[:, sl]
        v_ref[h] = vb[:, sl]


def _attn_body(q_ref, k_ref, v_ref, o_ref, acc_ref, l_ref):
    qi = pl.program_id(1)
    acc_ref[...] = jnp.zeros_like(acc_ref)
    l_ref[...] = jnp.zeros_like(l_ref)
    CH = 1024
    for c in range(T // CH):
        @pl.when(qi >= c * (CH // TQ))
        def _():
            rows = qi * TQ + lax.broadcasted_iota(jnp.int32, (TQ, CH), 0)
            cols = c * CH + lax.broadcasted_iota(jnp.int32, (TQ, CH), 1)
            future = cols > rows
            for s_idx in range(2):
                q = q_ref[s_idx]
                ks = k_ref[s_idx, pl.ds(c * CH, CH), :]
                vs = v_ref[s_idx, pl.ds(c * CH, CH), :]
                s = lax.dot_general(q, ks, (((1,), (1,)), ((), ())),
                                    preferred_element_type=F32)
                p = jnp.where(future, 0.0, jnp.exp(s))
                l_ref[:, s_idx:s_idx + 1] = (
                    l_ref[:, s_idx:s_idx + 1]
                    + jnp.sum(p, axis=-1, keepdims=True))
                av = lax.dot_general(p.astype(BF), vs,
                                     (((1,), (0,)), ((), ())),
                                     preferred_element_type=F32)
                sl = slice(s_idx * HD, (s_idx + 1) * HD)
                acc_ref[:, sl] = acc_ref[:, sl] + av
    for s_idx in range(2):
        sl = slice(s_idx * HD, (s_idx + 1) * HD)
        o_ref[:, sl] = ((acc_ref[:, sl] / l_ref[:, s_idx:s_idx + 1])
                        .astype(BF))


def _post_body(a_ref, x_ref, wo_ref, bo_ref, g2_ref, gw_ref,
               x1_ref, xn2_ref, rt_ref, wob_ref):
    i = pl.program_id(0)

    @pl.when(i == 0)
    def _():
        wob_ref[...] = wo_ref[...].astype(BF)

    a = a_ref[...]
    o = lax.dot_general(a, wob_ref[...], (((1,), (1,)), ((), ())),
                        preferred_element_type=F32)
    x1 = o + bo_ref[...] + x_ref[...]
    x1_ref[...] = x1
    ms = jnp.mean(x1 * x1, axis=-1, keepdims=True)
    xn2 = x1 * lax.rsqrt(ms + 1e-6) * g2_ref[...]
    xn2_ref[...] = xn2
    lg = lax.dot_general(xn2, gw_ref[...], (((1,), (1,)), ((), ())),
                         preferred_element_type=F32)
    mx = jnp.max(lg, axis=-1, keepdims=True)
    ex = jnp.exp(lg - mx)
    p = ex / jnp.sum(ex, axis=-1, keepdims=True)
    colsE = lax.broadcasted_iota(jnp.int32, (TQ, E), 1)
    v1 = jnp.max(p, axis=-1, keepdims=True)
    i1 = jnp.min(jnp.where(p == v1, colsE, E), axis=-1, keepdims=True)
    p2 = jnp.where(colsE == i1, -1.0, p)
    v2 = jnp.max(p2, axis=-1, keepdims=True)
    i2 = jnp.min(jnp.where(p2 == v2, colsE, E), axis=-1, keepdims=True)
    rt_ref[...] = jnp.concatenate(
        [v1, v2, i1.astype(F32), i2.astype(F32),
         jnp.zeros((TQ, 4), F32)], axis=-1)


def _ffn_body(eids_ref, used_ref, xs_ref, w1_ref, w3_ref, w2_ref,
              b1_ref, b3_ref, b2_ref, o_ref,
              w1b_ref, w3b_ref, w2b_ref):
    i = pl.program_id(0)
    prev = eids_ref[jnp.maximum(i - 1, 0)]
    changed = jnp.logical_or(i == 0, eids_ref[i] != prev)

    @pl.when(changed)
    def _():
        w1b_ref[...] = w1_ref[0].astype(BF)
        w3b_ref[...] = w3_ref[0].astype(BF)
        w2b_ref[...] = w2_ref[0].astype(BF)

    @pl.when(used_ref[i] > 0)
    def _():
        xb = xs_ref[...].astype(BF)
        h1 = lax.dot_general(xb, w1b_ref[...], (((1,), (1,)), ((), ())),
                             preferred_element_type=F32) + b1_ref[0]
        h3 = lax.dot_general(xb, w3b_ref[...], (((1,), (1,)), ((), ())),
                             preferred_element_type=F32) + b3_ref[0]
        h = ((h1 * lax.logistic(h1)) * h3).astype(BF)
        o = lax.dot_general(h, w2b_ref[...], (((1,), (1,)), ((), ())),
                            preferred_element_type=F32) + b2_ref[0]
        o_ref[...] = o


def _final_body(x1_ref, a_ref, b_ref, rt_ref, o_ref):
    o_ref[...] = (x1_ref[...]
                  + rt_ref[:, 0:1] * a_ref[...]
                  + rt_ref[:, 1:2] * b_ref[...])


def _sc_mesh():
    return plsc.VectorSubcoreMesh(core_axis_name="c", subcore_axis_name="s")


def _sc_dispatch_scatter(xn2, pt):
    per_w = T // NW

    @functools.partial(
        pl.kernel,
        out_type=jax.ShapeDtypeStruct((NPOS, D), F32),
        mesh=_sc_mesh(),
        scratch_types=[
            pltpu.VMEM((per_w,), jnp.int32),
            pltpu.VMEM((per_w,), jnp.int32),
            pltpu.VMEM((per_w, D), F32),
            pltpu.SemaphoreType.DMA,
        ],
    )
    def k(tab_hbm, idx_hbm, out_hbm, i0_v, i1_v, rows_v, sem):
        wid = lax.axis_index("s") * NC + lax.axis_index("c")
        base = wid * per_w
        pltpu.sync_copy(idx_hbm.at[pl.ds(base, per_w)], i0_v)
        pltpu.sync_copy(idx_hbm.at[pl.ds(T + base, per_w)], i1_v)
        pltpu.async_copy(tab_hbm.at[pl.ds(base, per_w)], rows_v, sem).wait()
        pltpu.sync_copy(rows_v, out_hbm.at[i0_v])
        pltpu.sync_copy(rows_v, out_hbm.at[i1_v])

    return k(xn2, pt)


def _sc_gather_rows(table, indices, nrows, chunk):
    per_w = nrows // NW

    @functools.partial(
        pl.kernel,
        out_type=jax.ShapeDtypeStruct((nrows, D), F32),
        mesh=_sc_mesh(),
        scratch_types=[
            pltpu.VMEM((chunk,), jnp.int32),
            pltpu.VMEM((chunk, D), F32),
            pltpu.SemaphoreType.DMA,
        ],
    )
    def k(tab_hbm, idx_hbm, out_hbm, idx_v, rows_v, sem):
        wid = lax.axis_index("s") * NC + lax.axis_index("c")
        base = wid * per_w

        @pl.loop(0, per_w // chunk)
        def _(c):
            off = base + c * chunk
            pltpu.sync_copy(idx_hbm.at[pl.ds(off, chunk)], idx_v)
            pltpu.async_copy(tab_hbm.at[idx_v], rows_v, sem).wait()
            pltpu.sync_copy(rows_v, out_hbm.at[pl.ds(off, chunk)])

    return k(table, indices)


def _routing_meta(route):
    vals = route[:, :K]
    idx = route[:, K:2 * K].astype(jnp.int32)
    e_flat = idx.reshape(-1)
    val_flat = vals.reshape(-1)
    oh = (e_flat[:, None] == jnp.arange(E, dtype=jnp.int32)[None, :])
    oh = oh.astype(jnp.int32)
    z = oh
    for s in (1, 16, 256):
        z = sum(jnp.pad(z, ((m * s, 0), (0, 0)))[:T * K]
                for m in range(1, 16)) + z
    ranks = z - oh
    r = jnp.sum(ranks * oh, axis=1)
    counts = jnp.sum(oh, axis=0)
    padded = ((counts + TILE - 1) // TILE) * TILE
    offs = jnp.concatenate(
        [jnp.zeros((1,), padded.dtype), jnp.cumsum(padded)[:-1]])
    P = (offs[e_flat] + r).astype(jnp.int32)
    del val_flat
    cum = jnp.cumsum(padded)
    tile_starts = jnp.arange(NTILES, dtype=cum.dtype) * TILE
    eids = jnp.minimum(
        jnp.searchsorted(cum, tile_starts, side='right'), E - 1)
    eids = eids.astype(jnp.int32)
    used = (tile_starts < offs[eids] + counts[eids]).astype(jnp.int32)
    pt = P.reshape(T, K).T.reshape(K * T)
    return eids, used, pt


def kernel(x, g1, g2, Wqkv, bqkv, Wout, bout, gateW, w1, b1, w2, b2, w3, b3):
    xf = x.reshape(T, D)
    Wg = Wqkv.reshape(H, 3, HD, D).transpose(1, 0, 2, 3)
    bg = bqkv.reshape(H, 3, HD).transpose(1, 0, 2)
    sgn = jnp.concatenate(
        [-jnp.ones((HALF, 1), F32), jnp.ones((HALF, 1), F32)])
    swp = jnp.concatenate([jnp.arange(HALF, HD), jnp.arange(0, HALF)])
    Wswap = Wg[:2, :, swp, :] * sgn
    bswap = bg[:2, :, swp] * sgn[:, 0]
    W5 = jnp.concatenate([Wg.reshape(3 * D, D),
                          Wswap.reshape(2 * D, D)])
    b5 = jnp.concatenate([bg.reshape(3 * D),
                          bswap.reshape(2 * D)]).reshape(1, 5 * D)

    q4, k4, v4 = pl.pallas_call(
        _prelude_body,
        grid=(T // TQ,),
        in_specs=[
            pl.BlockSpec((TQ, D), lambda i: (i, 0)),
            pl.BlockSpec((1, D), lambda i: (0, 0)),
            pl.BlockSpec((5 * D, D), lambda i: (0, 0)),
            pl.BlockSpec((1, 5 * D), lambda i: (0, 0)),
        ],
        out_specs=[
            pl.BlockSpec((H, TQ, HD), lambda i: (0, i, 0)),
            pl.BlockSpec((H, TQ, HD), lambda i: (0, i, 0)),
            pl.BlockSpec((H, TQ, HD), lambda i: (0, i, 0)),
        ],
        out_shape=[
            jax.ShapeDtypeStruct((H, T, HD), BF),
            jax.ShapeDtypeStruct((H, T, HD), BF),
            jax.ShapeDtypeStruct((H, T, HD), BF),
        ],
        scratch_shapes=[pltpu.VMEM((5 * D, D), BF)],
    )(xf, g1.reshape(1, D), W5, b5)

    attn_t = pl.pallas_call(
        _attn_body,
        grid=(H // 2, T // TQ),
        in_specs=[
            pl.BlockSpec((2, TQ, HD), lambda h, i: (h, i, 0)),
            pl.BlockSpec((2, T, HD), lambda h, i: (h, 0, 0)),
            pl.BlockSpec((2, T, HD), lambda h, i: (h, 0, 0)),
        ],
        out_specs=pl.BlockSpec((TQ, 2 * HD), lambda h, i: (i, h)),
        out_shape=jax.ShapeDtypeStruct((T, D), BF),
        scratch_shapes=[
            pltpu.VMEM((TQ, 2 * HD), F32),
            pltpu.VMEM((TQ, 128), F32),
        ],
    )(q4, k4, v4)

    x1, xn2, route = pl.pallas_call(
        _post_body,
        grid=(T // TQ,),
        in_specs=[
            pl.BlockSpec((TQ, D), lambda i: (i, 0)),
            pl.BlockSpec((TQ, D), lambda i: (i, 0)),
            pl.BlockSpec((D, D), lambda i: (0, 0)),
            pl.BlockSpec((1, D), lambda i: (0, 0)),
            pl.BlockSpec((1, D), lambda i: (0, 0)),
            pl.BlockSpec((E, D), lambda i: (0, 0)),
        ],
        out_specs=[
            pl.BlockSpec((TQ, D), lambda i: (i, 0)),
            pl.BlockSpec((TQ, D), lambda i: (i, 0)),
            pl.BlockSpec((TQ, E), lambda i: (i, 0)),
        ],
        out_shape=[
            jax.ShapeDtypeStruct((T, D), F32),
            jax.ShapeDtypeStruct((T, D), F32),
            jax.ShapeDtypeStruct((T, E), F32),
        ],
        scratch_shapes=[pltpu.VMEM((D, D), BF)],
    )(attn_t, xf, Wout, bout.reshape(1, D), g2.reshape(1, D),
      gateW)

    eids, used, pt = _routing_meta(route)

    xs = _sc_dispatch_scatter(xn2, pt)

    ys = pl.pallas_call(
        _ffn_body,
        grid_spec=pltpu.PrefetchScalarGridSpec(
            num_scalar_prefetch=2,
            grid=(NTILES,),
            in_specs=[
                pl.BlockSpec((TILE, D), lambda i, eids, used: (i, 0)),
                pl.BlockSpec((1, INTER, D),
                             lambda i, eids, used: (eids[i], 0, 0)),
                pl.BlockSpec((1, INTER, D),
                             lambda i, eids, used: (eids[i], 0, 0)),
                pl.BlockSpec((1, D, INTER),
                             lambda i, eids, used: (eids[i], 0, 0)),
                pl.BlockSpec((1, 1, INTER),
                             lambda i, eids, used: (eids[i], 0, 0)),
                pl.BlockSpec((1, 1, INTER),
                             lambda i, eids, used: (eids[i], 0, 0)),
                pl.BlockSpec((1, 1, D),
                             lambda i, eids, used: (eids[i], 0, 0)),
            ],
            out_specs=pl.BlockSpec((TILE, D), lambda i, eids, used: (i, 0)),
            scratch_shapes=[
                pltpu.VMEM((INTER, D), BF),
                pltpu.VMEM((INTER, D), BF),
                pltpu.VMEM((D, INTER), BF),
            ],
        ),
        out_shape=jax.ShapeDtypeStruct((NPOS, D), F32),
    )(eids, used, xs, w1, w3, w2,
      b1.reshape(E, 1, INTER), b3.reshape(E, 1, INTER), b2.reshape(E, 1, D))

    ab = _sc_gather_rows(ys, pt, 2 * T, 64)

    out = pl.pallas_call(
        _final_body,
        grid=(T // TQ,),
        in_specs=[
            pl.BlockSpec((TQ, D), lambda i: (i, 0)),
            pl.BlockSpec((TQ, D), lambda i: (i, 0)),
            pl.BlockSpec((TQ, D), lambda i: (i + T // TQ, 0)),
            pl.BlockSpec((TQ, E), lambda i: (i, 0)),
        ],
        out_specs=pl.BlockSpec((TQ, D), lambda i: (i, 0)),
        out_shape=jax.ShapeDtypeStruct((T, D), F32),
    )(x1, ab, ab, route)
    return out.reshape(B, T, D)


# speedup vs baseline: 2.4399x; 1.0425x over previous
import functools

import jax
import jax.numpy as jnp
from jax import lax
from jax.experimental import pallas as pl
from jax.experimental.pallas import tpu as pltpu
from jax.experimental.pallas import tpu_sc as plsc

B, T, D = 1, 2048, 1024
H = 16
HD = D // H
HALF = HD // 2
E = 8
K = 2
INTER = 1024
SCALE = D ** (-0.5)

TQ = 256
TK = 256
NKT = T // TK
TILE = 128
NTILES = (T * K + E * (TILE - 1)) // TILE + 1
NPOS = NTILES * TILE

NC, NS = 2, 16
NW = NC * NS

BF = jnp.bfloat16
F32 = jnp.float32


def _prelude_body(x_ref, g1_ref, w5_ref, b5_ref, q_ref, k_ref, v_ref,
                  w5b_ref):
    i = pl.program_id(0)

    @pl.when(i == 0)
    def _():
        w5b_ref[...] = w5_ref[...].astype(BF)

    xb = x_ref[...]
    ms = jnp.mean(xb * xb, axis=-1, keepdims=True)
    xn = (xb * lax.rsqrt(ms + 1e-6) * g1_ref[...]).astype(BF)
    proj = lax.dot_general(xn, w5b_ref[...], (((1,), (1,)), ((), ())),
                           preferred_element_type=F32) + b5_ref[...]
    q, k, v = proj[:, :D], proj[:, D:2 * D], proj[:, 2 * D:3 * D]
    qs, ks = proj[:, 3 * D:4 * D], proj[:, 4 * D:]
    pos = (i * TQ
           + lax.broadcasted_iota(jnp.int32, (TQ, 128), 0)).astype(F32)
    lane = lax.broadcasted_iota(jnp.int32, (TQ, 128), 1) % HALF
    theta = jnp.exp(lane.astype(F32) * (-jnp.log(10000.0) / HALF))
    freq = pos * theta
    cosw = jnp.tile(jnp.cos(freq), (1, D // 128))
    sinw = jnp.tile(jnp.sin(freq), (1, D // 128))
    rq = ((q * cosw + qs * sinw) * SCALE).astype(BF)
    rk = (k * cosw + ks * sinw).astype(BF)
    vb = v.astype(BF)
    for h in range(H):
        sl = slice(h * HD, (h + 1) * HD)
        q_ref[h] = rq[:, sl]
        k_ref[h] = rk[:, sl]
        v_ref[h] = vb[:, sl]


def _attn_body(q_ref, k_ref, v_ref, o_ref, acc_ref, l_ref):
    qi = pl.program_id(1)
    acc_ref[...] = jnp.zeros_like(acc_ref)
    l_ref[...] = jnp.zeros_like(l_ref)
    CH = 1024
    for c in range(T // CH):
        @pl.when(qi >= c * (CH // TQ))
        def _():
            rows = qi * TQ + lax.broadcasted_iota(jnp.int32, (TQ, CH), 0)
            cols = c * CH + lax.broadcasted_iota(jnp.int32, (TQ, CH), 1)
            future = cols > rows
            for s_idx in range(2):
                q = q_ref[s_idx]
                ks = k_ref[s_idx, pl.ds(c * CH, CH), :]
                vs = v_ref[s_idx, pl.ds(c * CH, CH), :]
                s = lax.dot_general(q, ks, (((1,), (1,)), ((), ())),
                                    preferred_element_type=F32)
                p = jnp.where(future, 0.0, jnp.exp(s))
                l_ref[:, s_idx:s_idx + 1] = (
                    l_ref[:, s_idx:s_idx + 1]
                    + jnp.sum(p, axis=-1, keepdims=True))
                av = lax.dot_general(p.astype(BF), vs,
                                     (((1,), (0,)), ((), ())),
                                     preferred_element_type=F32)
                sl = slice(s_idx * HD, (s_idx + 1) * HD)
                acc_ref[:, sl] = acc_ref[:, sl] + av
    for s_idx in range(2):
        sl = slice(s_idx * HD, (s_idx + 1) * HD)
        o_ref[:, sl] = ((acc_ref[:, sl] / l_ref[:, s_idx:s_idx + 1])
                        .astype(BF))


def _post_body(a_ref, x_ref, wo_ref, bo_ref, g2_ref, gw_ref,
               x1_ref, xn2_ref, rt_ref, wob_ref):
    i = pl.program_id(0)

    @pl.when(i == 0)
    def _():
        wob_ref[...] = wo_ref[...].astype(BF)

    a = a_ref[...]
    o = lax.dot_general(a, wob_ref[...], (((1,), (1,)), ((), ())),
                        preferred_element_type=F32)
    x1 = o + bo_ref[...] + x_ref[...]
    x1_ref[...] = x1
    ms = jnp.mean(x1 * x1, axis=-1, keepdims=True)
    xn2 = x1 * lax.rsqrt(ms + 1e-6) * g2_ref[...]
    xn2_ref[...] = xn2
    lg = lax.dot_general(xn2, gw_ref[...], (((1,), (1,)), ((), ())),
                         preferred_element_type=F32)
    mx = jnp.max(lg, axis=-1, keepdims=True)
    ex = jnp.exp(lg - mx)
    p = ex / jnp.sum(ex, axis=-1, keepdims=True)
    colsE = lax.broadcasted_iota(jnp.int32, (TQ, E), 1)
    v1 = jnp.max(p, axis=-1, keepdims=True)
    i1 = jnp.min(jnp.where(p == v1, colsE, E), axis=-1, keepdims=True)
    p2 = jnp.where(colsE == i1, -1.0, p)
    v2 = jnp.max(p2, axis=-1, keepdims=True)
    i2 = jnp.min(jnp.where(p2 == v2, colsE, E), axis=-1, keepdims=True)
    rt_ref[...] = jnp.concatenate(
        [v1, v2, i1.astype(F32), i2.astype(F32),
         jnp.zeros((TQ, 4), F32)], axis=-1)


def _ffn_body(eids_ref, used_ref, xs_ref, w1_ref, w3_ref, w2_ref,
              b1_ref, b3_ref, b2_ref, o_ref,
              w1b_ref, w3b_ref, w2b_ref):
    i = pl.program_id(0)
    prev = eids_ref[jnp.maximum(i - 1, 0)]
    changed = jnp.logical_or(i == 0, eids_ref[i] != prev)

    @pl.when(changed)
    def _():
        w1b_ref[...] = w1_ref[0].astype(BF)
        w3b_ref[...] = w3_ref[0].astype(BF)
        w2b_ref[...] = w2_ref[0].astype(BF)

    @pl.when(used_ref[i] > 0)
    def _():
        xb = xs_ref[...].astype(BF)
        h1 = lax.dot_general(xb, w1b_ref[...], (((1,), (1,)), ((), ())),
                             preferred_element_type=F32) + b1_ref[0]
        h3 = lax.dot_general(xb, w3b_ref[...], (((1,), (1,)), ((), ())),
                             preferred_element_type=F32) + b3_ref[0]
        h = ((h1 * lax.logistic(h1)) * h3).astype(BF)
        o = lax.dot_general(h, w2b_ref[...], (((1,), (1,)), ((), ())),
                            preferred_element_type=F32) + b2_ref[0]
        o_ref[...] = o


def _final_body(x1_ref, a_ref, b_ref, rt_ref, o_ref):
    o_ref[...] = (x1_ref[...]
                  + rt_ref[:, 0:1] * a_ref[...]
                  + rt_ref[:, 1:2] * b_ref[...])


def _sc_mesh():
    return plsc.VectorSubcoreMesh(core_axis_name="c", subcore_axis_name="s")


def _sc_dispatch_scatter(xn2, pt):
    per_w = T // NW

    @functools.partial(
        pl.kernel,
        out_type=jax.ShapeDtypeStruct((NPOS, D), F32),
        mesh=_sc_mesh(),
        scratch_types=[
            pltpu.VMEM((per_w,), jnp.int32),
            pltpu.VMEM((per_w,), jnp.int32),
            pltpu.VMEM((per_w, D), F32),
            pltpu.SemaphoreType.DMA,
        ],
    )
    def k(tab_hbm, idx_hbm, out_hbm, i0_v, i1_v, rows_v, sem):
        wid = lax.axis_index("s") * NC + lax.axis_index("c")
        base = wid * per_w
        pltpu.sync_copy(idx_hbm.at[pl.ds(base, per_w)], i0_v)
        pltpu.sync_copy(idx_hbm.at[pl.ds(T + base, per_w)], i1_v)
        pltpu.async_copy(tab_hbm.at[pl.ds(base, per_w)], rows_v, sem).wait()
        pltpu.sync_copy(rows_v, out_hbm.at[i0_v])
        pltpu.sync_copy(rows_v, out_hbm.at[i1_v])

    return k(xn2, pt)


def _sc_gather_rows(table, indices, nrows, chunk):
    per_w = nrows // NW

    @functools.partial(
        pl.kernel,
        out_type=jax.ShapeDtypeStruct((nrows, D), F32),
        mesh=_sc_mesh(),
        scratch_types=[
            pltpu.VMEM((chunk,), jnp.int32),
            pltpu.VMEM((chunk, D), F32),
            pltpu.SemaphoreType.DMA,
        ],
    )
    def k(tab_hbm, idx_hbm, out_hbm, idx_v, rows_v, sem):
        wid = lax.axis_index("s") * NC + lax.axis_index("c")
        base = wid * per_w

        @pl.loop(0, per_w // chunk)
        def _(c):
            off = base + c * chunk
            pltpu.sync_copy(idx_hbm.at[pl.ds(off, chunk)], idx_v)
            pltpu.async_copy(tab_hbm.at[idx_v], rows_v, sem).wait()
            pltpu.sync_copy(rows_v, out_hbm.at[pl.ds(off, chunk)])

    return k(table, indices)


def _routing_meta(route):
    vals = route[:, :K]
    idx = route[:, K:2 * K].astype(jnp.int32)
    e_flat = idx.reshape(-1)
    val_flat = vals.reshape(-1)
    oh = (e_flat[:, None] == jnp.arange(E, dtype=jnp.int32)[None, :])
    del val_flat
    NBLK, BLK = 16, (T * K) // 16
    ohf = oh.astype(F32).reshape(NBLK, BLK, E)
    tril_s = (jnp.arange(BLK)[:, None] > jnp.arange(BLK)[None, :]
              ).astype(F32)
    intra = jnp.einsum('ij,bjk->bik', tril_s, ohf,
                       preferred_element_type=F32)
    bsums = jnp.sum(ohf, axis=1)
    trilb = (jnp.arange(NBLK)[:, None] > jnp.arange(NBLK)[None, :]
             ).astype(F32)
    carry = trilb @ bsums
    ranks = (intra + carry[:, None, :]).reshape(T * K, E)
    r = jnp.sum(ranks * ohf.reshape(T * K, E), axis=1).astype(jnp.int32)
    counts = jnp.sum(oh, axis=0).astype(jnp.int32)
    padded = ((counts + TILE - 1) // TILE) * TILE
    lt = (jnp.arange(E)[:, None] > jnp.arange(E)[None, :]).astype(jnp.int32)
    offs = lt @ padded
    cum = offs + padded
    P = (offs[e_flat] + r).astype(jnp.int32)
    tile_starts = jnp.arange(NTILES, dtype=jnp.int32) * TILE
    eids = jnp.minimum(
        jnp.sum(tile_starts[:, None] >= cum[None, :], axis=1), E - 1)
    eids = eids.astype(jnp.int32)
    used = (tile_starts < offs[eids] + counts[eids]).astype(jnp.int32)
    pt = P.reshape(T, K).T.reshape(K * T)
    return eids, used, pt


def kernel(x, g1, g2, Wqkv, bqkv, Wout, bout, gateW, w1, b1, w2, b2, w3, b3):
    xf = x.reshape(T, D)
    Wg = Wqkv.reshape(H, 3, HD, D).transpose(1, 0, 2, 3)
    bg = bqkv.reshape(H, 3, HD).transpose(1, 0, 2)
    sgn = jnp.concatenate(
        [-jnp.ones((HALF, 1), F32), jnp.ones((HALF, 1), F32)])
    swp = jnp.concatenate([jnp.arange(HALF, HD), jnp.arange(0, HALF)])
    Wswap = Wg[:2, :, swp, :] * sgn
    bswap = bg[:2, :, swp] * sgn[:, 0]
    W5 = jnp.concatenate([Wg.reshape(3 * D, D),
                          Wswap.reshape(2 * D, D)])
    b5 = jnp.concatenate([bg.reshape(3 * D),
                          bswap.reshape(2 * D)]).reshape(1, 5 * D)

    q4, k4, v4 = pl.pallas_call(
        _prelude_body,
        grid=(T // TQ,),
        in_specs=[
            pl.BlockSpec((TQ, D), lambda i: (i, 0)),
            pl.BlockSpec((1, D), lambda i: (0, 0)),
            pl.BlockSpec((5 * D, D), lambda i: (0, 0)),
            pl.BlockSpec((1, 5 * D), lambda i: (0, 0)),
        ],
        out_specs=[
            pl.BlockSpec((H, TQ, HD), lambda i: (0, i, 0)),
            pl.BlockSpec((H, TQ, HD), lambda i: (0, i, 0)),
            pl.BlockSpec((H, TQ, HD), lambda i: (0, i, 0)),
        ],
        out_shape=[
            jax.ShapeDtypeStruct((H, T, HD), BF),
            jax.ShapeDtypeStruct((H, T, HD), BF),
            jax.ShapeDtypeStruct((H, T, HD), BF),
        ],
        scratch_shapes=[pltpu.VMEM((5 * D, D), BF)],
    )(xf, g1.reshape(1, D), W5, b5)

    attn_t = pl.pallas_call(
        _attn_body,
        grid=(H // 2, T // TQ),
        in_specs=[
            pl.BlockSpec((2, TQ, HD), lambda h, i: (h, i, 0)),
            pl.BlockSpec((2, T, HD), lambda h, i: (h, 0, 0)),
            pl.BlockSpec((2, T, HD), lambda h, i: (h, 0, 0)),
        ],
        out_specs=pl.BlockSpec((TQ, 2 * HD), lambda h, i: (i, h)),
        out_shape=jax.ShapeDtypeStruct((T, D), BF),
        scratch_shapes=[
            pltpu.VMEM((TQ, 2 * HD), F32),
            pltpu.VMEM((TQ, 128), F32),
        ],
    )(q4, k4, v4)

    x1, xn2, route = pl.pallas_call(
        _post_body,
        grid=(T // TQ,),
        in_specs=[
            pl.BlockSpec((TQ, D), lambda i: (i, 0)),
            pl.BlockSpec((TQ, D), lambda i: (i, 0)),
            pl.BlockSpec((D, D), lambda i: (0, 0)),
            pl.BlockSpec((1, D), lambda i: (0, 0)),
            pl.BlockSpec((1, D), lambda i: (0, 0)),
            pl.BlockSpec((E, D), lambda i: (0, 0)),
        ],
        out_specs=[
            pl.BlockSpec((TQ, D), lambda i: (i, 0)),
            pl.BlockSpec((TQ, D), lambda i: (i, 0)),
            pl.BlockSpec((TQ, E), lambda i: (i, 0)),
        ],
        out_shape=[
            jax.ShapeDtypeStruct((T, D), F32),
            jax.ShapeDtypeStruct((T, D), F32),
            jax.ShapeDtypeStruct((T, E), F32),
        ],
        scratch_shapes=[pltpu.VMEM((D, D), BF)],
    )(attn_t, xf, Wout, bout.reshape(1, D), g2.reshape(1, D),
      gateW)

    eids, used, pt = _routing_meta(route)

    xs = _sc_dispatch_scatter(xn2, pt)

    ys = pl.pallas_call(
        _ffn_body,
        grid_spec=pltpu.PrefetchScalarGridSpec(
            num_scalar_prefetch=2,
            grid=(NTILES,),
            in_specs=[
                pl.BlockSpec((TILE, D), lambda i, eids, used: (i, 0)),
                pl.BlockSpec((1, INTER, D),
                             lambda i, eids, used: (eids[i], 0, 0)),
                pl.BlockSpec((1, INTER, D),
                             lambda i, eids, used: (eids[i], 0, 0)),
                pl.BlockSpec((1, D, INTER),
                             lambda i, eids, used: (eids[i], 0, 0)),
                pl.BlockSpec((1, 1, INTER),
                             lambda i, eids, used: (eids[i], 0, 0)),
                pl.BlockSpec((1, 1, INTER),
                             lambda i, eids, used: (eids[i], 0, 0)),
                pl.BlockSpec((1, 1, D),
                             lambda i, eids, used: (eids[i], 0, 0)),
            ],
            out_specs=pl.BlockSpec((TILE, D), lambda i, eids, used: (i, 0)),
            scratch_shapes=[
                pltpu.VMEM((INTER, D), BF),
                pltpu.VMEM((INTER, D), BF),
                pltpu.VMEM((D, INTER), BF),
            ],
        ),
        out_shape=jax.ShapeDtypeStruct((NPOS, D), F32),
    )(eids, used, xs, w1, w3, w2,
      b1.reshape(E, 1, INTER), b3.reshape(E, 1, INTER), b2.reshape(E, 1, D))

    ab = _sc_gather_rows(ys, pt, 2 * T, 64)

    out = pl.pallas_call(
        _final_body,
        grid=(T // TQ,),
        in_specs=[
            pl.BlockSpec((TQ, D), lambda i: (i, 0)),
            pl.BlockSpec((TQ, D), lambda i: (i, 0)),
            pl.BlockSpec((TQ, D), lambda i: (i + T // TQ, 0)),
            pl.BlockSpec((TQ, E), lambda i: (i, 0)),
        ],
        out_specs=pl.BlockSpec((TQ, D), lambda i: (i, 0)),
        out_shape=jax.ShapeDtypeStruct((T, D), F32),
    )(x1, ab, ab, route)
    return out.reshape(B, T, D)


# speedup vs baseline: 2.5421x; 1.0419x over previous
import functools

import jax
import jax.numpy as jnp
from jax import lax
from jax.experimental import pallas as pl
from jax.experimental.pallas import tpu as pltpu
from jax.experimental.pallas import tpu_sc as plsc

B, T, D = 1, 2048, 1024
H = 16
HD = D // H
HALF = HD // 2
E = 8
K = 2
INTER = 1024
SCALE = D ** (-0.5)

TQ = 256
TK = 256
NKT = T // TK
TILE = 128
NTILES = (T * K + E * (TILE - 1)) // TILE + 1
NPOS = NTILES * TILE

NC, NS = 2, 16
NW = NC * NS

BF = jnp.bfloat16
F32 = jnp.float32


def _prelude_body(x_ref, g1_ref, w5_ref, b5_ref, q_ref, k_ref, v_ref,
                  w5b_ref):
    i = pl.program_id(0)

    @pl.when(i == 0)
    def _():
        w5b_ref[...] = w5_ref[...].astype(BF)

    xb = x_ref[...]
    ms = jnp.mean(xb * xb, axis=-1, keepdims=True)
    xn = (xb * lax.rsqrt(ms + 1e-6) * g1_ref[...]).astype(BF)
    proj = lax.dot_general(xn, w5b_ref[...], (((1,), (1,)), ((), ())),
                           preferred_element_type=F32) + b5_ref[...]
    q, k, v = proj[:, :D], proj[:, D:2 * D], proj[:, 2 * D:3 * D]
    qs, ks = proj[:, 3 * D:4 * D], proj[:, 4 * D:]
    pos = (i * TQ
           + lax.broadcasted_iota(jnp.int32, (TQ, 128), 0)).astype(F32)
    lane = lax.broadcasted_iota(jnp.int32, (TQ, 128), 1) % HALF
    theta = jnp.exp(lane.astype(F32) * (-jnp.log(10000.0) / HALF))
    freq = pos * theta
    cosw = jnp.tile(jnp.cos(freq), (1, D // 128))
    sinw = jnp.tile(jnp.sin(freq), (1, D // 128))
    rq = ((q * cosw + qs * sinw) * SCALE).astype(BF)
    rk = (k * cosw + ks * sinw).astype(BF)
    vb = v.astype(BF)
    for h in range(H):
        sl = slice(h * HD, (h + 1) * HD)
        q_ref[h] = rq[:, sl]
        k_ref[h] = rk[:, sl]
        v_ref[h] = vb[:, sl]


def _attn_body(q_ref, k_ref, v_ref, o_ref, acc_ref, l_ref):
    qi = pl.program_id(1)
    acc_ref[...] = jnp.zeros_like(acc_ref)
    l_ref[...] = jnp.zeros_like(l_ref)
    CH = 1024
    for c in range(T // CH):
        @pl.when(qi >= c * (CH // TQ))
        def _():
            rows = qi * TQ + lax.broadcasted_iota(jnp.int32, (TQ, CH), 0)
            cols = c * CH + lax.broadcasted_iota(jnp.int32, (TQ, CH), 1)
            future = cols > rows
            for s_idx in range(2):
                q = q_ref[s_idx]
                ks = k_ref[s_idx, pl.ds(c * CH, CH), :]
                vs = v_ref[s_idx, pl.ds(c * CH, CH), :]
                s = lax.dot_general(q, ks, (((1,), (1,)), ((), ())),
                                    preferred_element_type=F32)
                p = jnp.where(future, jnp.bfloat16(0.0),
                              jnp.exp(s.astype(BF)))
                l_ref[:, s_idx:s_idx + 1] = (
                    l_ref[:, s_idx:s_idx + 1]
                    + jnp.sum(p, axis=-1, keepdims=True, dtype=F32))
                av = lax.dot_general(p, vs,
                                     (((1,), (0,)), ((), ())),
                                     preferred_element_type=F32)
                sl = slice(s_idx * HD, (s_idx + 1) * HD)
                acc_ref[:, sl] = acc_ref[:, sl] + av
    for s_idx in range(2):
        sl = slice(s_idx * HD, (s_idx + 1) * HD)
        o_ref[:, sl] = ((acc_ref[:, sl] / l_ref[:, s_idx:s_idx + 1])
                        .astype(BF))


def _post_body(a_ref, x_ref, wo_ref, bo_ref, g2_ref, gw_ref,
               x1_ref, xn2_ref, rt_ref, wob_ref):
    i = pl.program_id(0)

    @pl.when(i == 0)
    def _():
        wob_ref[...] = wo_ref[...].astype(BF)

    a = a_ref[...]
    o = lax.dot_general(a, wob_ref[...], (((1,), (1,)), ((), ())),
                        preferred_element_type=F32)
    x1 = o + bo_ref[...] + x_ref[...]
    x1_ref[...] = x1
    ms = jnp.mean(x1 * x1, axis=-1, keepdims=True)
    xn2 = x1 * lax.rsqrt(ms + 1e-6) * g2_ref[...]
    xn2_ref[...] = xn2
    lg = lax.dot_general(xn2, gw_ref[...], (((1,), (1,)), ((), ())),
                         preferred_element_type=F32)
    mx = jnp.max(lg, axis=-1, keepdims=True)
    ex = jnp.exp(lg - mx)
    p = ex / jnp.sum(ex, axis=-1, keepdims=True)
    colsE = lax.broadcasted_iota(jnp.int32, (TQ, E), 1)
    v1 = jnp.max(p, axis=-1, keepdims=True)
    i1 = jnp.min(jnp.where(p == v1, colsE, E), axis=-1, keepdims=True)
    p2 = jnp.where(colsE == i1, -1.0, p)
    v2 = jnp.max(p2, axis=-1, keepdims=True)
    i2 = jnp.min(jnp.where(p2 == v2, colsE, E), axis=-1, keepdims=True)
    rt_ref[...] = jnp.concatenate(
        [v1, v2, i1.astype(F32), i2.astype(F32),
         jnp.zeros((TQ, 4), F32)], axis=-1)


def _ffn_body(eids_ref, used_ref, xs_ref, w1_ref, w3_ref, w2_ref,
              b1_ref, b3_ref, b2_ref, o_ref,
              w1b_ref, w3b_ref, w2b_ref):
    i = pl.program_id(0)
    prev = eids_ref[jnp.maximum(i - 1, 0)]
    changed = jnp.logical_or(i == 0, eids_ref[i] != prev)

    @pl.when(changed)
    def _():
        w1b_ref[...] = w1_ref[0].astype(BF)
        w3b_ref[...] = w3_ref[0].astype(BF)
        w2b_ref[...] = w2_ref[0].astype(BF)

    @pl.when(used_ref[i] > 0)
    def _():
        xb = xs_ref[...].astype(BF)
        h1 = lax.dot_general(xb, w1b_ref[...], (((1,), (1,)), ((), ())),
                             preferred_element_type=F32) + b1_ref[0]
        h3 = lax.dot_general(xb, w3b_ref[...], (((1,), (1,)), ((), ())),
                             preferred_element_type=F32) + b3_ref[0]
        h = ((h1 * lax.logistic(h1)) * h3).astype(BF)
        o = lax.dot_general(h, w2b_ref[...], (((1,), (1,)), ((), ())),
                            preferred_element_type=F32) + b2_ref[0]
        o_ref[...] = o


def _final_body(x1_ref, a_ref, b_ref, rt_ref, o_ref):
    o_ref[...] = (x1_ref[...]
                  + rt_ref[:, 0:1] * a_ref[...]
                  + rt_ref[:, 1:2] * b_ref[...])


def _sc_mesh():
    return plsc.VectorSubcoreMesh(core_axis_name="c", subcore_axis_name="s")


def _sc_dispatch_scatter(xn2, pt):
    per_w = T // NW

    @functools.partial(
        pl.kernel,
        out_type=jax.ShapeDtypeStruct((NPOS, D), F32),
        mesh=_sc_mesh(),
        scratch_types=[
            pltpu.VMEM((per_w,), jnp.int32),
            pltpu.VMEM((per_w,), jnp.int32),
            pltpu.VMEM((per_w, D), F32),
            pltpu.SemaphoreType.DMA,
        ],
    )
    def k(tab_hbm, idx_hbm, out_hbm, i0_v, i1_v, rows_v, sem):
        wid = lax.axis_index("s") * NC + lax.axis_index("c")
        base = wid * per_w
        pltpu.sync_copy(idx_hbm.at[pl.ds(base, per_w)], i0_v)
        pltpu.sync_copy(idx_hbm.at[pl.ds(T + base, per_w)], i1_v)
        pltpu.async_copy(tab_hbm.at[pl.ds(base, per_w)], rows_v, sem).wait()
        pltpu.sync_copy(rows_v, out_hbm.at[i0_v])
        pltpu.sync_copy(rows_v, out_hbm.at[i1_v])

    return k(xn2, pt)


def _sc_gather_rows(table, indices, nrows, chunk):
    per_w = nrows // NW

    @functools.partial(
        pl.kernel,
        out_type=jax.ShapeDtypeStruct((nrows, D), F32),
        mesh=_sc_mesh(),
        scratch_types=[
            pltpu.VMEM((chunk,), jnp.int32),
            pltpu.VMEM((chunk, D), F32),
            pltpu.SemaphoreType.DMA,
        ],
    )
    def k(tab_hbm, idx_hbm, out_hbm, idx_v, rows_v, sem):
        wid = lax.axis_index("s") * NC + lax.axis_index("c")
        base = wid * per_w

        @pl.loop(0, per_w // chunk)
        def _(c):
            off = base + c * chunk
            pltpu.sync_copy(idx_hbm.at[pl.ds(off, chunk)], idx_v)
            pltpu.async_copy(tab_hbm.at[idx_v], rows_v, sem).wait()
            pltpu.sync_copy(rows_v, out_hbm.at[pl.ds(off, chunk)])

    return k(table, indices)


def _routing_meta(route):
    vals = route[:, :K]
    idx = route[:, K:2 * K].astype(jnp.int32)
    e_flat = idx.reshape(-1)
    val_flat = vals.reshape(-1)
    oh = (e_flat[:, None] == jnp.arange(E, dtype=jnp.int32)[None, :])
    del val_flat
    NBLK, BLK = 16, (T * K) // 16
    ohf = oh.astype(F32).reshape(NBLK, BLK, E)
    tril_s = (jnp.arange(BLK)[:, None] > jnp.arange(BLK)[None, :]
              ).astype(F32)
    intra = jnp.einsum('ij,bjk->bik', tril_s, ohf,
                       preferred_element_type=F32)
    bsums = jnp.sum(ohf, axis=1)
    trilb = (jnp.arange(NBLK)[:, None] > jnp.arange(NBLK)[None, :]
             ).astype(F32)
    carry = trilb @ bsums
    ranks = (intra + carry[:, None, :]).reshape(T * K, E)
    r = jnp.sum(ranks * ohf.reshape(T * K, E), axis=1).astype(jnp.int32)
    counts = jnp.sum(oh, axis=0).astype(jnp.int32)
    padded = ((counts + TILE - 1) // TILE) * TILE
    lt = (jnp.arange(E)[:, None] > jnp.arange(E)[None, :]).astype(jnp.int32)
    offs = lt @ padded
    cum = offs + padded
    P = (offs[e_flat] + r).astype(jnp.int32)
    tile_starts = jnp.arange(NTILES, dtype=jnp.int32) * TILE
    eids = jnp.minimum(
        jnp.sum(tile_starts[:, None] >= cum[None, :], axis=1), E - 1)
    eids = eids.astype(jnp.int32)
    used = (tile_starts < offs[eids] + counts[eids]).astype(jnp.int32)
    pt = P.reshape(T, K).T.reshape(K * T)
    return eids, used, pt


def kernel(x, g1, g2, Wqkv, bqkv, Wout, bout, gateW, w1, b1, w2, b2, w3, b3):
    xf = x.reshape(T, D)
    Wg = Wqkv.reshape(H, 3, HD, D).transpose(1, 0, 2, 3)
    bg = bqkv.reshape(H, 3, HD).transpose(1, 0, 2)
    sgn = jnp.concatenate(
        [-jnp.ones((HALF, 1), F32), jnp.ones((HALF, 1), F32)])
    swp = jnp.concatenate([jnp.arange(HALF, HD), jnp.arange(0, HALF)])
    Wswap = Wg[:2, :, swp, :] * sgn
    bswap = bg[:2, :, swp] * sgn[:, 0]
    W5 = jnp.concatenate([Wg.reshape(3 * D, D),
                          Wswap.reshape(2 * D, D)])
    b5 = jnp.concatenate([bg.reshape(3 * D),
                          bswap.reshape(2 * D)]).reshape(1, 5 * D)

    q4, k4, v4 = pl.pallas_call(
        _prelude_body,
        grid=(T // TQ,),
        in_specs=[
            pl.BlockSpec((TQ, D), lambda i: (i, 0)),
            pl.BlockSpec((1, D), lambda i: (0, 0)),
            pl.BlockSpec((5 * D, D), lambda i: (0, 0)),
            pl.BlockSpec((1, 5 * D), lambda i: (0, 0)),
        ],
        out_specs=[
            pl.BlockSpec((H, TQ, HD), lambda i: (0, i, 0)),
            pl.BlockSpec((H, TQ, HD), lambda i: (0, i, 0)),
            pl.BlockSpec((H, TQ, HD), lambda i: (0, i, 0)),
        ],
        out_shape=[
            jax.ShapeDtypeStruct((H, T, HD), BF),
            jax.ShapeDtypeStruct((H, T, HD), BF),
            jax.ShapeDtypeStruct((H, T, HD), BF),
        ],
        scratch_shapes=[pltpu.VMEM((5 * D, D), BF)],
    )(xf, g1.reshape(1, D), W5, b5)

    attn_t = pl.pallas_call(
        _attn_body,
        grid=(H // 2, T // TQ),
        in_specs=[
            pl.BlockSpec((2, TQ, HD), lambda h, i: (h, i, 0)),
            pl.BlockSpec((2, T, HD), lambda h, i: (h, 0, 0)),
            pl.BlockSpec((2, T, HD), lambda h, i: (h, 0, 0)),
        ],
        out_specs=pl.BlockSpec((TQ, 2 * HD), lambda h, i: (i, h)),
        out_shape=jax.ShapeDtypeStruct((T, D), BF),
        scratch_shapes=[
            pltpu.VMEM((TQ, 2 * HD), F32),
            pltpu.VMEM((TQ, 128), F32),
        ],
    )(q4, k4, v4)

    x1, xn2, route = pl.pallas_call(
        _post_body,
        grid=(T // TQ,),
        in_specs=[
            pl.BlockSpec((TQ, D), lambda i: (i, 0)),
            pl.BlockSpec((TQ, D), lambda i: (i, 0)),
            pl.BlockSpec((D, D), lambda i: (0, 0)),
            pl.BlockSpec((1, D), lambda i: (0, 0)),
            pl.BlockSpec((1, D), lambda i: (0, 0)),
            pl.BlockSpec((E, D), lambda i: (0, 0)),
        ],
        out_specs=[
            pl.BlockSpec((TQ, D), lambda i: (i, 0)),
            pl.BlockSpec((TQ, D), lambda i: (i, 0)),
            pl.BlockSpec((TQ, E), lambda i: (i, 0)),
        ],
        out_shape=[
            jax.ShapeDtypeStruct((T, D), F32),
            jax.ShapeDtypeStruct((T, D), F32),
            jax.ShapeDtypeStruct((T, E), F32),
        ],
        scratch_shapes=[pltpu.VMEM((D, D), BF)],
    )(attn_t, xf, Wout, bout.reshape(1, D), g2.reshape(1, D),
      gateW)

    eids, used, pt = _routing_meta(route)

    xs = _sc_dispatch_scatter(xn2, pt)

    ys = pl.pallas_call(
        _ffn_body,
        grid_spec=pltpu.PrefetchScalarGridSpec(
            num_scalar_prefetch=2,
            grid=(NTILES,),
            in_specs=[
                pl.BlockSpec((TILE, D), lambda i, eids, used: (i, 0)),
                pl.BlockSpec((1, INTER, D),
                             lambda i, eids, used: (eids[i], 0, 0)),
                pl.BlockSpec((1, INTER, D),
                             lambda i, eids, used: (eids[i], 0, 0)),
                pl.BlockSpec((1, D, INTER),
                             lambda i, eids, used: (eids[i], 0, 0)),
                pl.BlockSpec((1, 1, INTER),
                             lambda i, eids, used: (eids[i], 0, 0)),
                pl.BlockSpec((1, 1, INTER),
                             lambda i, eids, used: (eids[i], 0, 0)),
                pl.BlockSpec((1, 1, D),
                             lambda i, eids, used: (eids[i], 0, 0)),
            ],
            out_specs=pl.BlockSpec((TILE, D), lambda i, eids, used: (i, 0)),
            scratch_shapes=[
                pltpu.VMEM((INTER, D), BF),
                pltpu.VMEM((INTER, D), BF),
                pltpu.VMEM((D, INTER), BF),
            ],
        ),
        out_shape=jax.ShapeDtypeStruct((NPOS, D), F32),
    )(eids, used, xs, w1, w3, w2,
      b1.reshape(E, 1, INTER), b3.reshape(E, 1, INTER), b2.reshape(E, 1, D))

    ab = _sc_gather_rows(ys, pt, 2 * T, 64)

    out = pl.pallas_call(
        _final_body,
        grid=(T // TQ,),
        in_specs=[
            pl.BlockSpec((TQ, D), lambda i: (i, 0)),
            pl.BlockSpec((TQ, D), lambda i: (i, 0)),
            pl.BlockSpec((TQ, D), lambda i: (i + T // TQ, 0)),
            pl.BlockSpec((TQ, E), lambda i: (i, 0)),
        ],
        out_specs=pl.BlockSpec((TQ, D), lambda i: (i, 0)),
        out_shape=jax.ShapeDtypeStruct((T, D), F32),
    )(x1, ab, ab, route)
    return out.reshape(B, T, D)
